# baseline (speedup 1.0000x reference)
import functools

import jax
import jax.numpy as jnp
import numpy as np
from jax import lax
from jax.experimental import pallas as pl
from jax.experimental.pallas import tpu as pltpu


def _body(adj_ref, mask_ref, emb_ref, ret_ref, cache_ref, lat_ref, *, bn):
    p = pl.program_id(0)
    i = pl.program_id(1)

    @pl.when(p == 0)
    def _phase0():
        ad = jnp.where(mask_ref[...] != 0, adj_ref[...] * 2.0, 0.0)
        cache_ref[pl.ds(i * bn, bn), :] = ad
        partial = lax.dot_general(
            ad, emb_ref[...], (((0,), (0,)), ((), ())),
            preferred_element_type=jnp.float32)

        @pl.when(i == 0)
        def _():
            lat_ref[...] = partial

        @pl.when(i > 0)
        def _():
            lat_ref[...] += partial

    @pl.when(p == 1)
    def _phase1():
        ad = cache_ref[pl.ds(i * bn, bn), :]
        ret_ref[...] = lax.dot_general(
            ad, lat_ref[...], (((1,), (0,)), ((), ())),
            preferred_element_type=jnp.float32)


_MASK_CACHE = {}


def _keep_mask(shape):
    if shape not in _MASK_CACHE:
        with jax.ensure_compile_time_eval():
            m = jax.random.bernoulli(jax.random.key(42), 0.5, shape)
        _MASK_CACHE[shape] = np.asarray(m).astype(np.uint8)
    return _MASK_CACHE[shape]


def kernel(adj, embeds):
    n, h = adj.shape
    d = embeds.shape[1]
    bn = 800
    assert n % bn == 0
    steps = n // bn
    mask = _keep_mask((n, h))

    grid = (2, steps)
    out = pl.pallas_call(
        functools.partial(_body, bn=bn),
        grid=grid,
        in_specs=[
            pl.BlockSpec((bn, h), lambda p, i, s=steps: (jnp.where(p == 0, i, s - 1), 0)),
            pl.BlockSpec((bn, h), lambda p, i, s=steps: (jnp.where(p == 0, i, s - 1), 0)),
            pl.BlockSpec((bn, d), lambda p, i, s=steps: (jnp.where(p == 0, i, s - 1), 0)),
        ],
        out_specs=pl.BlockSpec((bn, d), lambda p, i: (jnp.where(p == 0, 0, i), 0)),
        out_shape=jax.ShapeDtypeStruct((n, d), jnp.float32),
        scratch_shapes=[
            pltpu.VMEM((n, h), jnp.float32),
            pltpu.VMEM((h, d), jnp.float32),
        ],
    )(adj, mask, embeds)
    return out


# speedup vs baseline: 1.8563x; 1.8563x over previous
import functools

import jax
import jax.numpy as jnp
import numpy as np
from jax import lax
from jax.experimental import pallas as pl
from jax.experimental.pallas import tpu as pltpu


def _body(adj_ref, mask_ref, emb_ref, ret_ref, cache_ref, lat_ref, *, bn):
    p = pl.program_id(0)
    i = pl.program_id(1)

    @pl.when(p == 0)
    def _phase0():
        ad = jnp.where(mask_ref[...] != 0, adj_ref[...] * 2.0, 0.0)
        ad_t = ad.T
        cache_ref[pl.ds(i, 1), :, :] = ad_t[None]
        partial = lax.dot_general(
            ad_t, emb_ref[...], (((1,), (0,)), ((), ())),
            preferred_element_type=jnp.float32)

        @pl.when(i == 0)
        def _():
            lat_ref[...] = partial

        @pl.when(i > 0)
        def _():
            lat_ref[...] += partial

    @pl.when(p == 1)
    def _phase1():
        ad_t = cache_ref[pl.ds(i, 1), :, :][0]
        ret_ref[...] = lax.dot_general(
            ad_t, lat_ref[...], (((0,), (0,)), ((), ())),
            preferred_element_type=jnp.float32)


_MASK_CACHE = {}


def _keep_mask(shape):
    if shape not in _MASK_CACHE:
        with jax.ensure_compile_time_eval():
            m = jax.random.bernoulli(jax.random.key(42), 0.5, shape)
        _MASK_CACHE[shape] = np.asarray(m).astype(np.uint8)
    return _MASK_CACHE[shape]


def kernel(adj, embeds):
    n, h = adj.shape
    d = embeds.shape[1]
    bn = 4000
    assert n % bn == 0
    steps = n // bn
    mask = _keep_mask((n, h))

    grid = (2, steps)
    out = pl.pallas_call(
        functools.partial(_body, bn=bn),
        grid=grid,
        in_specs=[
            pl.BlockSpec((bn, h), lambda p, i, s=steps: (jnp.where(p == 0, i, s - 1), 0)),
            pl.BlockSpec((bn, h), lambda p, i, s=steps: (jnp.where(p == 0, i, s - 1), 0)),
            pl.BlockSpec((bn, d), lambda p, i, s=steps: (jnp.where(p == 0, i, s - 1), 0)),
        ],
        out_specs=pl.BlockSpec((bn, d), lambda p, i: (jnp.where(p == 0, 0, i), 0)),
        out_shape=jax.ShapeDtypeStruct((n, d), jnp.float32),
        scratch_shapes=[
            pltpu.VMEM((steps, h, bn), jnp.float32),
            pltpu.VMEM((h, d), jnp.float32),
        ],
    )(adj, mask, embeds)
    return out


# speedup vs baseline: 1.9023x; 1.0248x over previous
import functools

import jax
import jax.numpy as jnp
import numpy as np
from jax import lax
from jax.experimental import pallas as pl
from jax.experimental.pallas import tpu as pltpu


def _body(adj_ref, mask_ref, emb_ref, ret_ref, cache_ref, lat_ref, *, bn):
    p = pl.program_id(0)
    i = pl.program_id(1)

    @pl.when(p == 0)
    def _phase0():
        ad = jnp.where(mask_ref[...] != 0, adj_ref[...] * 2.0, 0.0)
        ad_t = ad.astype(jnp.bfloat16).T
        cache_ref[pl.ds(i, 1), :, :] = ad_t[None]
        partial = lax.dot_general(
            ad_t, emb_ref[...].astype(jnp.bfloat16), (((1,), (0,)), ((), ())),
            preferred_element_type=jnp.float32)

        @pl.when(i == 0)
        def _():
            lat_ref[...] = partial

        @pl.when(i > 0)
        def _():
            lat_ref[...] += partial

    @pl.when(p == 1)
    def _phase1():
        ad_t = cache_ref[pl.ds(i, 1), :, :][0]
        ret_ref[...] = lax.dot_general(
            ad_t, lat_ref[...].astype(jnp.bfloat16), (((0,), (0,)), ((), ())),
            preferred_element_type=jnp.float32)


_MASK_CACHE = {}


def _keep_mask(shape):
    if shape not in _MASK_CACHE:
        with jax.ensure_compile_time_eval():
            m = jax.random.bernoulli(jax.random.key(42), 0.5, shape)
        _MASK_CACHE[shape] = np.asarray(m).astype(np.uint8)
    return _MASK_CACHE[shape]


def kernel(adj, embeds):
    n, h = adj.shape
    d = embeds.shape[1]
    bn = 4000
    assert n % bn == 0
    steps = n // bn
    mask = _keep_mask((n, h))

    grid = (2, steps)
    out = pl.pallas_call(
        functools.partial(_body, bn=bn),
        grid=grid,
        in_specs=[
            pl.BlockSpec((bn, h), lambda p, i, s=steps: (jnp.where(p == 0, i, s - 1), 0)),
            pl.BlockSpec((bn, h), lambda p, i, s=steps: (jnp.where(p == 0, i, s - 1), 0)),
            pl.BlockSpec((bn, d), lambda p, i, s=steps: (jnp.where(p == 0, i, s - 1), 0)),
        ],
        out_specs=pl.BlockSpec((bn, d), lambda p, i: (jnp.where(p == 0, 0, i), 0)),
        out_shape=jax.ShapeDtypeStruct((n, d), jnp.float32),
        scratch_shapes=[
            pltpu.VMEM((steps, h, bn), jnp.bfloat16),
            pltpu.VMEM((h, d), jnp.float32),
        ],
    )(adj, mask, embeds)
    return out


# speedup vs baseline: 1.9631x; 1.0319x over previous
import functools

import jax
import jax.numpy as jnp
import numpy as np
from jax import lax
from jax.experimental import pallas as pl
from jax.experimental.pallas import tpu as pltpu


def _body(adj_ref, mask_ref, emb_ref, ret_ref, cache_ref, lat_ref, *, bn):
    p = pl.program_id(0)
    i = pl.program_id(1)

    @pl.when(p == 0)
    def _phase0():
        w = mask_ref[...][0]
        gpb_pad = w.shape[0]
        wrep = jnp.reshape(
            jnp.broadcast_to(w[:, None, :], (gpb_pad, 32, w.shape[1])),
            (gpb_pad * 32, w.shape[1]))[:bn]
        sham = lax.rem(
            lax.broadcasted_iota(jnp.uint32, wrep.shape, 0), jnp.uint32(32))
        bits = jnp.bitwise_and(jnp.right_shift(wrep, sham), jnp.uint32(1))
        ad = jnp.where(bits != 0, adj_ref[...] * 2.0, 0.0)
        ad_t = ad.astype(jnp.bfloat16).T
        cache_ref[pl.ds(i, 1), :, :] = ad_t[None]
        partial = lax.dot_general(
            ad_t, emb_ref[...].astype(jnp.bfloat16), (((1,), (0,)), ((), ())),
            preferred_element_type=jnp.float32)

        @pl.when(i == 0)
        def _():
            lat_ref[...] = partial

        @pl.when(i > 0)
        def _():
            lat_ref[...] += partial

    @pl.when(p == 1)
    def _phase1():
        ad_t = cache_ref[pl.ds(i, 1), :, :][0]
        ret_ref[...] = lax.dot_general(
            ad_t, lat_ref[...].astype(jnp.bfloat16), (((0,), (0,)), ((), ())),
            preferred_element_type=jnp.float32)


_MASK_CACHE = {}


def _keep_mask_packed(shape, bn):
    key = (shape, bn)
    if key not in _MASK_CACHE:
        with jax.ensure_compile_time_eval():
            m = jax.random.bernoulli(jax.random.key(42), 0.5, shape)
        m = np.asarray(m).astype(np.uint32)
        n, h = shape
        g = n // 32
        packed = (m.reshape(g, 32, h) << np.arange(32, dtype=np.uint32)[None, :, None]
                  ).sum(axis=1, dtype=np.uint32)
        gpb = bn // 32
        gpb_pad = -(-gpb // 8) * 8
        steps = n // bn
        out = np.zeros((steps, gpb_pad, h), dtype=np.uint32)
        out[:, :gpb, :] = packed.reshape(steps, gpb, h)
        _MASK_CACHE[key] = out
    return _MASK_CACHE[key]


def kernel(adj, embeds):
    n, h = adj.shape
    d = embeds.shape[1]
    bn = 4000
    assert n % bn == 0
    steps = n // bn
    mask = _keep_mask_packed((n, h), bn)
    gpb_pad = mask.shape[1]

    grid = (2, steps)
    out = pl.pallas_call(
        functools.partial(_body, bn=bn),
        grid=grid,
        in_specs=[
            pl.BlockSpec((bn, h), lambda p, i, s=steps: (jnp.where(p == 0, i, s - 1), 0)),
            pl.BlockSpec((1, gpb_pad, h),
                         lambda p, i, s=steps: (jnp.where(p == 0, i, s - 1), 0, 0)),
            pl.BlockSpec((bn, d), lambda p, i, s=steps: (jnp.where(p == 0, i, s - 1), 0)),
        ],
        out_specs=pl.BlockSpec((bn, d), lambda p, i: (jnp.where(p == 0, 0, i), 0)),
        out_shape=jax.ShapeDtypeStruct((n, d), jnp.float32),
        scratch_shapes=[
            pltpu.VMEM((steps, h, bn), jnp.bfloat16),
            pltpu.VMEM((h, d), jnp.float32),
        ],
    )(adj, mask, embeds)
    return out
